# baseline (speedup 1.0000x reference)
import functools

import jax
import jax.numpy as jnp
from jax import lax
from jax.experimental import pallas as pl
from jax.experimental.pallas import tpu as pltpu
from jax.experimental.pallas import tpu_sc as plsc


def _sc_ctx_sum(xT, emb_table, n_workers=32, num_cores=2):
    ctx, batch = xT.shape
    _, d = emb_table.shape
    nb = batch // n_workers

    mesh = plsc.VectorSubcoreMesh(core_axis_name="c", subcore_axis_name="s")

    @functools.partial(
        pl.kernel,
        out_type=jax.ShapeDtypeStruct((batch, d), jnp.float32),
        mesh=mesh,
        scratch_types=[
            pltpu.VMEM((ctx, nb), jnp.int32),
            pltpu.VMEM((nb, d), jnp.float32),
            pltpu.SemaphoreType.DMA,
        ],
    )
    def sc_sum(xT_hbm, table_hbm, out_hbm, idx_v, acc_v, sem):
        wid = lax.axis_index("s") * num_cores + lax.axis_index("c")
        base = wid * nb
        pltpu.sync_copy(xT_hbm.at[:, pl.ds(base, nb)], idx_v)
        pltpu.async_copy(table_hbm.at[idx_v.at[0]], acc_v, sem).wait()
        adds = [
            pltpu.async_copy(table_hbm.at[idx_v.at[c]], acc_v, sem, add=True)
            for c in range(1, ctx)
        ]
        for cp in adds:
            cp.wait()
        pltpu.sync_copy(acc_v, out_hbm.at[pl.ds(base, nb)])

    return sc_sum(xT, emb_table)


def _tc_project_t(sums_t_bf16, W, bcol, ctx, vt=1536):
    d, batch = sums_t_bf16.shape
    vocab = W.shape[0]
    inv_ctx = 1.0 / ctx

    def body(s_ref, w_ref, b_ref, o_ref):
        w = (w_ref[...] * inv_ctx).astype(jnp.bfloat16)
        o_ref[...] = lax.dot_general(
            w, s_ref[...], (((1,), (0,)), ((), ())),
            preferred_element_type=jnp.float32,
        ) + b_ref[...].T

    return pl.pallas_call(
        body,
        grid=(pl.cdiv(vocab, vt),),
        in_specs=[
            pl.BlockSpec((d, batch), lambda j: (0, 0)),
            pl.BlockSpec((vt, d), lambda j: (j, 0)),
            pl.BlockSpec((1, vt), lambda j: (0, j)),
        ],
        out_specs=pl.BlockSpec((vt, batch), lambda j: (j, 0)),
        out_shape=jax.ShapeDtypeStruct((vocab, batch), jnp.float32),
        compiler_params=pltpu.CompilerParams(
            vmem_limit_bytes=100 * 1024 * 1024,
        ),
    )(sums_t_bf16, W, bcol)


def kernel(x, emb_table, W, b):
    ctx = x.shape[1]
    sums = _sc_ctx_sum(x.T, emb_table)
    logits_t = _tc_project_t(
        sums.T.astype(jnp.bfloat16), W, b.reshape(1, -1), ctx)
    return logits_t.T
